# Initial kernel scaffold; baseline (speedup 1.0000x reference)
import jax
import jax.numpy as jnp
from jax.experimental import pallas as pl


def kernel(obs, hete_pick, W1, b1, W2, b2, Wc1, bc1, Wc2, bc2):
    raise NotImplementedError("write your pallas kernel here")


# speedup vs baseline: 1.9391x; 1.9391x over previous
import functools

import jax
import jax.numpy as jnp
from jax.experimental import pallas as pl
from jax.experimental.pallas import tpu as pltpu

E = 8
D = 2048
F = 4096
A = 32
T = 256
N_TOK = 8192
P = N_TOK + E * T
NT = P // T
TC = 256
NC = N_TOK // TC


def _expert_body(exp_ref, x_ref, w1_ref, b1_ref, w2_ref, b2_ref, o_ref):
    h = jnp.dot(x_ref[...], w1_ref[0], preferred_element_type=jnp.float32)
    h = jnp.maximum(h + b1_ref[0], 0.0).astype(jnp.bfloat16)
    y = jnp.dot(h, w2_ref[0], preferred_element_type=jnp.float32)
    o_ref[...] = y + b2_ref[0]


def _expert_matmul(expert_of_tile, xs, W1, b1, W2, b2):
    grid_spec = pltpu.PrefetchScalarGridSpec(
        num_scalar_prefetch=1,
        grid=(NT,),
        in_specs=[
            pl.BlockSpec((T, D), lambda t, exp: (t, 0)),
            pl.BlockSpec((1, D, F), lambda t, exp: (exp[t], 0, 0)),
            pl.BlockSpec((1, 1, F), lambda t, exp: (exp[t], 0, 0)),
            pl.BlockSpec((1, F, A), lambda t, exp: (exp[t], 0, 0)),
            pl.BlockSpec((1, 1, A), lambda t, exp: (exp[t], 0, 0)),
        ],
        out_specs=pl.BlockSpec((T, A), lambda t, exp: (t, 0)),
    )
    return pl.pallas_call(
        _expert_body,
        grid_spec=grid_spec,
        out_shape=jax.ShapeDtypeStruct((P, A), jnp.float32),
    )(expert_of_tile, xs, W1.astype(jnp.bfloat16), b1.reshape(E, 1, F),
      W2.astype(jnp.bfloat16), b2.reshape(E, 1, A))


def _critic_body(x_ref, wc1_ref, bc1_ref, wc2_ref, bc2_ref, o_ref):
    h = jnp.dot(x_ref[...], wc1_ref[...], preferred_element_type=jnp.float32)
    h = jnp.maximum(h + bc1_ref[...], 0.0).astype(jnp.bfloat16)
    v = jnp.dot(h, wc2_ref[...], preferred_element_type=jnp.float32)
    o_ref[...] = v + bc2_ref[...]


def _critic(x, Wc1, bc1, Wc2, bc2):
    return pl.pallas_call(
        _critic_body,
        grid=(NC,),
        in_specs=[
            pl.BlockSpec((TC, D), lambda t: (t, 0)),
            pl.BlockSpec((D, F), lambda t: (0, 0)),
            pl.BlockSpec((1, F), lambda t: (0, 0)),
            pl.BlockSpec((F, 1), lambda t: (0, 0)),
            pl.BlockSpec((1, 1), lambda t: (0, 0)),
        ],
        out_specs=pl.BlockSpec((TC, 1), lambda t: (t, 0)),
        out_shape=jax.ShapeDtypeStruct((N_TOK, 1), jnp.float32),
    )(x, Wc1.astype(jnp.bfloat16), bc1.reshape(1, F),
      Wc2.astype(jnp.bfloat16), bc2.reshape(1, 1))


def kernel(obs, hete_pick, W1, b1, W2, b2, Wc1, bc1, Wc2, bc2):
    n_threads, n_agents, d = obs.shape
    x = obs.reshape(-1, d)
    pick = hete_pick.reshape(-1).astype(jnp.int32)

    onehot = (pick[:, None] == jnp.arange(E, dtype=jnp.int32)[None, :])
    counts = jnp.sum(onehot, axis=0, dtype=jnp.int32)
    padded = ((counts + T - 1) // T) * T
    starts = jnp.cumsum(padded) - padded
    ends = starts + padded
    rank = jnp.cumsum(onehot, axis=0, dtype=jnp.int32) - onehot
    pos = starts[pick] + jnp.take_along_axis(rank, pick[:, None], axis=1)[:, 0]
    tile_starts = jnp.arange(NT, dtype=jnp.int32) * T
    expert_of_tile = jnp.minimum(
        jnp.searchsorted(ends, tile_starts, side="right"), E - 1
    ).astype(jnp.int32)

    xb = x.astype(jnp.bfloat16)
    xs = jnp.zeros((P, d), jnp.bfloat16).at[pos].set(xb)

    ys = _expert_matmul(expert_of_tile, xs, W1, b1, W2, b2)
    logits = ys[pos].reshape(n_threads, n_agents, A)

    value = _critic(xb, Wc1, bc1, Wc2, bc2).reshape(n_threads, n_agents, 1)
    return logits, value
